# baseline (speedup 1.0000x reference)
import functools

import jax
import jax.numpy as jnp
from jax import lax
from jax.experimental import pallas as pl
from jax.experimental.pallas import tpu as pltpu
from jax.experimental.pallas import tpu_sc as plsc

N_CLS = 1000
N_CTX_ROWS = 16
SEQ_LEN = 77
OUT_LEN = 69
N_EMB = 53
D = 512
NW = 32
ITERS = (N_CLS + NW - 1) // NW


def _splice_body(emb_hbm, ctx_hbm, adj_hbm, out_hbm,
                 adj_v, emb_idx_v, dst_emb_v, dst_ctx_v,
                 buf_emb, buf_ctx, sem_g, sem_c, sem_s):
    cid = lax.axis_index("c")
    sid = lax.axis_index("s")
    wid = sid * 2 + cid

    pltpu.sync_copy(adj_hbm, adj_v)

    def per_class(t, carry_none):
        i = wid + t * NW

        @pl.when(i < N_CLS)
        def _():
            ls = [plsc.load_gather(adj_v, [jnp.full((16,), i * 4 + k, jnp.int32)])
                  for k in range(4)]
            l0, l1, l2, l3 = ls
            E = jnp.int32(SEQ_LEN)
            carry = jnp.int32(0)
            for c in range(5):
                jv = lax.iota(jnp.int32, 16) + jnp.int32(16 * c)
                valid = jv < OUT_LEN
                idx = jnp.where(jv >= l3 - 4, jv + 8,
                      jnp.where(jv >= l3 - 6, E + 14 + (jv - (l3 - 6)),
                      jnp.where(jv >= l2 - 2, jv + 6,
                      jnp.where(jv >= l2 - 4, E + 12 + (jv - (l2 - 4)),
                      jnp.where(jv >= l1,     jv + 4,
                      jnp.where(jv >= l1 - 2, E + 10 + (jv - (l1 - 2)),
                      jnp.where(jv >= l0 + 2, jv + 2,
                      jnp.where(jv >= l0,     E + 8 + (jv - l0),
                      jnp.where(jv >= 9,      jv,
                      jnp.where(jv >= 1,      E + (jv - 1),
                                jnp.int32(0)))))))))))
                is_emb = idx < E
                emb_valid = jnp.logical_and(is_emb, valid)
                ctx_valid = jnp.logical_and(jnp.logical_not(is_emb), valid)
                pos_emb = plsc.cumsum(emb_valid.astype(jnp.int32)) + carry - 1
                carry = carry + jnp.sum(emb_valid.astype(jnp.int32))
                out_row = i * OUT_LEN + jv
                plsc.store_scatter(emb_idx_v, [pos_emb], i * SEQ_LEN + idx,
                                   mask=emb_valid)
                plsc.store_scatter(dst_emb_v, [pos_emb], out_row, mask=emb_valid)
                plsc.store_scatter(dst_ctx_v, [idx - E], out_row, mask=ctx_valid)

            g = pltpu.async_copy(emb_hbm.at[emb_idx_v], buf_emb, sem_g)
            cc = pltpu.async_copy(
                ctx_hbm.at[pl.ds(i * N_CTX_ROWS, N_CTX_ROWS)], buf_ctx, sem_c)
            g.wait()
            cc.wait()
            s1 = pltpu.async_copy(buf_emb, out_hbm.at[dst_emb_v], sem_s)
            s2 = pltpu.async_copy(buf_ctx, out_hbm.at[dst_ctx_v], sem_c)
            s1.wait()
            s2.wait()

        return carry_none

    lax.fori_loop(0, ITERS, per_class, None)


@jax.jit
def _splice(emb_flat, ctx_flat, adj_flat):
    mesh = plsc.VectorSubcoreMesh(core_axis_name="c", subcore_axis_name="s")
    f = pl.kernel(
        _splice_body,
        out_type=jax.ShapeDtypeStruct((N_CLS * OUT_LEN, D), jnp.float32),
        mesh=mesh,
        compiler_params=pltpu.CompilerParams(needs_layout_passes=False),
        scratch_types=[
            pltpu.VMEM((N_CLS * 4,), jnp.int32),
            pltpu.VMEM((N_EMB,), jnp.int32),
            pltpu.VMEM((N_EMB,), jnp.int32),
            pltpu.VMEM((N_CTX_ROWS,), jnp.int32),
            pltpu.VMEM((N_EMB, D), jnp.float32),
            pltpu.VMEM((N_CTX_ROWS, D), jnp.float32),
            pltpu.SemaphoreType.DMA,
            pltpu.SemaphoreType.DMA,
            pltpu.SemaphoreType.DMA,
        ],
    )
    return f(emb_flat, ctx_flat, adj_flat)


def kernel(ctx, embedding, adj_locations):
    emb_flat = embedding.reshape(N_CLS * SEQ_LEN, D)
    ctx_flat = ctx.reshape(N_CLS * N_CTX_ROWS, D)
    adj_flat = adj_locations.reshape(N_CLS * 4)
    out = _splice(emb_flat, ctx_flat, adj_flat)
    return out.reshape(N_CLS, OUT_LEN, D)


# speedup vs baseline: 1.0113x; 1.0113x over previous
import functools

import jax
import jax.numpy as jnp
from jax import lax
from jax.experimental import pallas as pl
from jax.experimental.pallas import tpu as pltpu
from jax.experimental.pallas import tpu_sc as plsc

N_CLS = 1000
N_CTX_ROWS = 16
SEQ_LEN = 77
OUT_LEN = 69
N_EMB = 53
D = 512
NW = 32
PER_W = 32
PAIRS = PER_W // 2


def _compute_idx(adj_v, i, emb_idx_v, dst_emb_v, dst_ctx_v):
    ls = [plsc.load_gather(adj_v, [jnp.full((16,), i * 4 + k, jnp.int32)])
          for k in range(4)]
    l0, l1, l2, l3 = ls
    E = jnp.int32(SEQ_LEN)
    carry = jnp.int32(0)
    for c in range(5):
        jv = lax.iota(jnp.int32, 16) + jnp.int32(16 * c)
        valid = jv < OUT_LEN
        idx = jnp.where(jv >= l3 - 4, jv + 8,
              jnp.where(jv >= l3 - 6, E + 14 + (jv - (l3 - 6)),
              jnp.where(jv >= l2 - 2, jv + 6,
              jnp.where(jv >= l2 - 4, E + 12 + (jv - (l2 - 4)),
              jnp.where(jv >= l1,     jv + 4,
              jnp.where(jv >= l1 - 2, E + 10 + (jv - (l1 - 2)),
              jnp.where(jv >= l0 + 2, jv + 2,
              jnp.where(jv >= l0,     E + 8 + (jv - l0),
              jnp.where(jv >= 9,      jv,
              jnp.where(jv >= 1,      E + (jv - 1),
                        jnp.int32(0)))))))))))
        is_emb = idx < E
        emb_valid = jnp.logical_and(is_emb, valid)
        ctx_valid = jnp.logical_and(jnp.logical_not(is_emb), valid)
        pos_emb = plsc.cumsum(emb_valid.astype(jnp.int32)) + carry - 1
        carry = carry + jnp.sum(emb_valid.astype(jnp.int32))
        out_row = i * OUT_LEN + jv
        plsc.store_scatter(emb_idx_v, [pos_emb], i * SEQ_LEN + idx,
                           mask=emb_valid)
        plsc.store_scatter(dst_emb_v, [pos_emb], out_row, mask=emb_valid)
        plsc.store_scatter(dst_ctx_v, [idx - E], out_row, mask=ctx_valid)


def _splice_body(emb_hbm, ctx_hbm, adj_hbm, out_hbm,
                 adj_v,
                 emb_idx_a, dst_emb_a, dst_ctx_a, buf_emb_a, buf_ctx_a,
                 emb_idx_b, dst_emb_b, dst_ctx_b, buf_emb_b, buf_ctx_b,
                 sem_a, sem_b):
    cid = lax.axis_index("c")
    sid = lax.axis_index("s")
    wid = sid * 2 + cid
    base = wid * PER_W

    pltpu.sync_copy(adj_hbm, adj_v)

    sets = (
        (emb_idx_a, dst_emb_a, dst_ctx_a, buf_emb_a, buf_ctx_a, sem_a),
        (emb_idx_b, dst_emb_b, dst_ctx_b, buf_emb_b, buf_ctx_b, sem_b),
    )

    def cls_of(t):
        return jnp.minimum(base + t, N_CLS - 1)

    def issue_gather(t, s):
        emb_idx_v, _, _, buf_emb, buf_ctx, sem = s
        i = cls_of(t)
        _compute_idx(adj_v, i, emb_idx_v, s[1], s[2])
        g = pltpu.async_copy(emb_hbm.at[emb_idx_v], buf_emb, sem)
        c = pltpu.async_copy(
            ctx_hbm.at[pl.ds(i * N_CTX_ROWS, N_CTX_ROWS)], buf_ctx, sem)
        return g, c

    def wait_gather(t, s):
        emb_idx_v, _, _, buf_emb, buf_ctx, sem = s
        i = cls_of(t)
        pltpu.make_async_copy(emb_hbm.at[emb_idx_v], buf_emb, sem).wait()
        pltpu.make_async_copy(
            ctx_hbm.at[pl.ds(i * N_CTX_ROWS, N_CTX_ROWS)], buf_ctx, sem).wait()

    def issue_scatter(s):
        _, dst_emb_v, dst_ctx_v, buf_emb, buf_ctx, sem = s
        pltpu.async_copy(buf_emb, out_hbm.at[dst_emb_v], sem)
        pltpu.async_copy(buf_ctx, out_hbm.at[dst_ctx_v], sem)

    def wait_scatter(s):
        _, dst_emb_v, dst_ctx_v, buf_emb, buf_ctx, sem = s
        pltpu.make_async_copy(buf_emb, out_hbm.at[dst_emb_v], sem).wait()
        pltpu.make_async_copy(buf_ctx, out_hbm.at[dst_ctx_v], sem).wait()

    issue_gather(0, sets[0])
    issue_gather(1, sets[1])

    def pair_body(t2, carry_none):
        a = 2 * t2
        for h in range(2):
            s = sets[h]
            wait_gather(a + h, s)
            issue_scatter(s)
        for h in range(2):
            s = sets[h]
            wait_scatter(s)

            @pl.when(t2 < PAIRS - 1)
            def _():
                issue_gather(a + 2 + h, s)

        return carry_none

    lax.fori_loop(0, PAIRS, pair_body, None)


@jax.jit
def _splice(emb_flat, ctx_flat, adj_flat):
    mesh = plsc.VectorSubcoreMesh(core_axis_name="c", subcore_axis_name="s")
    f = pl.kernel(
        _splice_body,
        out_type=jax.ShapeDtypeStruct((N_CLS * OUT_LEN, D), jnp.float32),
        mesh=mesh,
        compiler_params=pltpu.CompilerParams(needs_layout_passes=False),
        scratch_types=[
            pltpu.VMEM((N_CLS * 4,), jnp.int32),
            pltpu.VMEM((N_EMB,), jnp.int32),
            pltpu.VMEM((N_EMB,), jnp.int32),
            pltpu.VMEM((N_CTX_ROWS,), jnp.int32),
            pltpu.VMEM((N_EMB, D), jnp.float32),
            pltpu.VMEM((N_CTX_ROWS, D), jnp.float32),
            pltpu.VMEM((N_EMB,), jnp.int32),
            pltpu.VMEM((N_EMB,), jnp.int32),
            pltpu.VMEM((N_CTX_ROWS,), jnp.int32),
            pltpu.VMEM((N_EMB, D), jnp.float32),
            pltpu.VMEM((N_CTX_ROWS, D), jnp.float32),
            pltpu.SemaphoreType.DMA,
            pltpu.SemaphoreType.DMA,
        ],
    )
    return f(emb_flat, ctx_flat, adj_flat)


def kernel(ctx, embedding, adj_locations):
    emb_flat = embedding.reshape(N_CLS * SEQ_LEN, D)
    ctx_flat = ctx.reshape(N_CLS * N_CTX_ROWS, D)
    adj_flat = adj_locations.reshape(N_CLS * 4)
    out = _splice(emb_flat, ctx_flat, adj_flat)
    return out.reshape(N_CLS, OUT_LEN, D)


# speedup vs baseline: 10.4779x; 10.3610x over previous
import functools

import jax
import jax.numpy as jnp
from jax import lax
from jax.experimental import pallas as pl
from jax.experimental.pallas import tpu as pltpu
from jax.experimental.pallas import tpu_sc as plsc

N_CLS = 1000
N_CTX_ROWS = 16
SEQ_LEN = 77
OUT_LEN = 69
N_EMB = 53
D = 512
NW = 32
PER_W = 32
PAIRS = PER_W // 2


def _compute_idx(adj_v, i, emb_idx_v, dst_emb_v, dst_ctx_v):
    ls = [plsc.load_gather(adj_v, [jnp.full((16,), k * N_CLS + i, jnp.int32)])
          for k in range(4)]
    l0, l1, l2, l3 = ls
    E = jnp.int32(SEQ_LEN)
    carry = jnp.int32(0)
    for c in range(5):
        jv = lax.iota(jnp.int32, 16) + jnp.int32(16 * c)
        valid = jv < OUT_LEN
        idx = jnp.where(jv >= l3 - 4, jv + 8,
              jnp.where(jv >= l3 - 6, E + 14 + (jv - (l3 - 6)),
              jnp.where(jv >= l2 - 2, jv + 6,
              jnp.where(jv >= l2 - 4, E + 12 + (jv - (l2 - 4)),
              jnp.where(jv >= l1,     jv + 4,
              jnp.where(jv >= l1 - 2, E + 10 + (jv - (l1 - 2)),
              jnp.where(jv >= l0 + 2, jv + 2,
              jnp.where(jv >= l0,     E + 8 + (jv - l0),
              jnp.where(jv >= 9,      jv,
              jnp.where(jv >= 1,      E + (jv - 1),
                        jnp.int32(0)))))))))))
        is_emb = idx < E
        emb_valid = jnp.logical_and(is_emb, valid)
        ctx_valid = jnp.logical_and(jnp.logical_not(is_emb), valid)
        pos_emb = plsc.cumsum(emb_valid.astype(jnp.int32)) + carry - 1
        carry = carry + jnp.sum(emb_valid.astype(jnp.int32))
        out_row = jv * N_CLS + i
        plsc.store_scatter(emb_idx_v, [pos_emb], idx * N_CLS + i,
                           mask=emb_valid)
        plsc.store_scatter(dst_emb_v, [pos_emb], out_row, mask=emb_valid)
        plsc.store_scatter(dst_ctx_v, [idx - E], out_row, mask=ctx_valid)


def _splice_body(emb_hbm, ctx_hbm, adj_hbm, out_hbm,
                 adj_v,
                 emb_idx_a, dst_emb_a, dst_ctx_a, buf_emb_a, buf_ctx_a,
                 emb_idx_b, dst_emb_b, dst_ctx_b, buf_emb_b, buf_ctx_b,
                 sem_a, sem_b):
    cid = lax.axis_index("c")
    sid = lax.axis_index("s")
    wid = sid * 2 + cid
    base = wid * PER_W

    pltpu.sync_copy(adj_hbm, adj_v)

    sets = (
        (emb_idx_a, dst_emb_a, dst_ctx_a, buf_emb_a, buf_ctx_a, sem_a),
        (emb_idx_b, dst_emb_b, dst_ctx_b, buf_emb_b, buf_ctx_b, sem_b),
    )

    def cls_of(t):
        return jnp.minimum(base + t, N_CLS - 1)

    def issue_gather(t, s):
        emb_idx_v, _, _, buf_emb, buf_ctx, sem = s
        i = cls_of(t)
        _compute_idx(adj_v, i, emb_idx_v, s[1], s[2])
        pltpu.async_copy(emb_hbm.at[emb_idx_v], buf_emb, sem)
        pltpu.async_copy(
            ctx_hbm.at[pl.ds(i * N_CTX_ROWS, N_CTX_ROWS)], buf_ctx, sem)

    def wait_gather(t, s):
        emb_idx_v, _, _, buf_emb, buf_ctx, sem = s
        i = cls_of(t)
        pltpu.make_async_copy(emb_hbm.at[emb_idx_v], buf_emb, sem).wait()
        pltpu.make_async_copy(
            ctx_hbm.at[pl.ds(i * N_CTX_ROWS, N_CTX_ROWS)], buf_ctx, sem).wait()

    def issue_scatter(s):
        _, dst_emb_v, dst_ctx_v, buf_emb, buf_ctx, sem = s
        pltpu.async_copy(buf_emb, out_hbm.at[dst_emb_v], sem)
        pltpu.async_copy(buf_ctx, out_hbm.at[dst_ctx_v], sem)

    def wait_scatter(s):
        _, dst_emb_v, dst_ctx_v, buf_emb, buf_ctx, sem = s
        pltpu.make_async_copy(buf_emb, out_hbm.at[dst_emb_v], sem).wait()
        pltpu.make_async_copy(buf_ctx, out_hbm.at[dst_ctx_v], sem).wait()

    issue_gather(0, sets[0])
    issue_gather(1, sets[1])

    def pair_body(t2, carry_none):
        a = 2 * t2
        for h in range(2):
            s = sets[h]
            wait_gather(a + h, s)
            issue_scatter(s)
        for h in range(2):
            s = sets[h]
            wait_scatter(s)

            @pl.when(t2 < PAIRS - 1)
            def _():
                issue_gather(a + 2 + h, s)

        return carry_none

    lax.fori_loop(0, PAIRS, pair_body, None)


@jax.jit
def _splice(emb_t_flat, ctx_flat, adj_t_flat):
    mesh = plsc.VectorSubcoreMesh(core_axis_name="c", subcore_axis_name="s")
    f = pl.kernel(
        _splice_body,
        out_type=jax.ShapeDtypeStruct((OUT_LEN * N_CLS, D), jnp.float32),
        mesh=mesh,
        compiler_params=pltpu.CompilerParams(needs_layout_passes=False),
        scratch_types=[
            pltpu.VMEM((N_CLS * 4,), jnp.int32),
            pltpu.VMEM((N_EMB,), jnp.int32),
            pltpu.VMEM((N_EMB,), jnp.int32),
            pltpu.VMEM((N_CTX_ROWS,), jnp.int32),
            pltpu.VMEM((N_EMB, D), jnp.float32),
            pltpu.VMEM((N_CTX_ROWS, D), jnp.float32),
            pltpu.VMEM((N_EMB,), jnp.int32),
            pltpu.VMEM((N_EMB,), jnp.int32),
            pltpu.VMEM((N_CTX_ROWS,), jnp.int32),
            pltpu.VMEM((N_EMB, D), jnp.float32),
            pltpu.VMEM((N_CTX_ROWS, D), jnp.float32),
            pltpu.SemaphoreType.DMA,
            pltpu.SemaphoreType.DMA,
        ],
    )
    return f(emb_t_flat, ctx_flat, adj_t_flat)


def kernel(ctx, embedding, adj_locations):
    emb_t_flat = embedding.transpose(1, 0, 2).reshape(SEQ_LEN * N_CLS, D)
    ctx_flat = ctx.reshape(N_CLS * N_CTX_ROWS, D)
    adj_t_flat = adj_locations.transpose(1, 0).reshape(4 * N_CLS)
    out = _splice(emb_t_flat, ctx_flat, adj_t_flat)
    return out.reshape(OUT_LEN, N_CLS, D).transpose(1, 0, 2)
